# baseline (speedup 1.0000x reference)
import math

import numpy as np
import jax
import jax.numpy as jnp
from jax.experimental import pallas as pl

_D_IN = 16
_N = 26
_SIZES = [_N - i for i in range(_N)]
_TOTAL = sum(_SIZES)
_OFFS = [int(v) for v in np.cumsum([0] + _SIZES)]
_LANES = 128
_BB = 1024

_POW2 = [1 << math.ceil(math.log2(k)) for k in _SIZES]
_R = [_LANES // p for p in _POW2]

_GROUPS = []
for _h in range(_N):
    if _GROUPS and _GROUPS[-1][0] == _R[_h]:
        _GROUPS[-1][1].append(_h)
    else:
        _GROUPS.append((_R[_h], [_h]))


def _body(*refs):
    ng = len(_GROUPS)
    x_refs = refs[:ng]
    w_refs = refs[ng:2 * ng]
    b_refs = refs[2 * ng:3 * ng]
    out_refs = refs[3 * ng:]
    for g, (r, heads) in enumerate(_GROUPS):
        y = jax.lax.dot_general(
            x_refs[g][...], w_refs[g][...], (((1,), (0,)), ((), ())),
            preferred_element_type=jnp.float32) + b_refs[g][...]
        for j, h in enumerate(heads):
            out_refs[h][...] = y[:, j * _LANES:(j + 1) * _LANES]


def kernel(x, W_cat, b_cat):
    B = x.shape[0]
    Wt = W_cat.T

    xs, ws, bs = [], [], []
    for r, heads in _GROUPS:
        xs.append(x.reshape(B // r, _D_IN * r))
        wblks, bblks = [], []
        for h in heads:
            k, off, p = _SIZES[h], _OFFS[h], _POW2[h]
            wpad = jnp.pad(Wt[:, off:off + k], ((0, 0), (0, p - k)))
            wblks.append(jnp.kron(jnp.eye(r, dtype=jnp.float32), wpad))
            bblks.append(jnp.tile(jnp.pad(b_cat[off:off + k], (0, p - k)), r))
        ws.append(jnp.concatenate(wblks, axis=1))
        bs.append(jnp.concatenate(bblks)[None, :])

    grid = (B // _BB,)
    in_specs, out_specs, out_shapes = [], [], []
    for (r, heads), w in zip(_GROUPS, ws):
        in_specs.append(
            pl.BlockSpec((_BB // r, _D_IN * r), lambda i: (i, 0)))
    for w in ws:
        in_specs.append(pl.BlockSpec(w.shape, lambda i: (0, 0)))
    for b in bs:
        in_specs.append(pl.BlockSpec(b.shape, lambda i: (0, 0)))
    for h in range(_N):
        r = _R[h]
        out_shapes.append(
            jax.ShapeDtypeStruct((B // r, _LANES), jnp.float32))
        out_specs.append(pl.BlockSpec((_BB // r, _LANES), lambda i: (i, 0)))

    outs = pl.pallas_call(
        _body,
        grid=grid,
        in_specs=in_specs,
        out_specs=out_specs,
        out_shape=out_shapes,
    )(*xs, *ws, *bs)

    final = []
    for h in range(_N):
        k, p = _SIZES[h], _POW2[h]
        o = outs[h].reshape(B, p)
        final.append(o if p == k else o[:, :k])
    return tuple(final)
